# baseline (speedup 1.0000x reference)
import functools

import jax
import jax.numpy as jnp
from jax import lax
from jax.experimental import pallas as pl
from jax.experimental.pallas import tpu as pltpu
from jax.experimental.pallas import tpu_sc as plsc

_N = 10000
_NC = 2
_NS = 16
_CH = 80


def _mlp_body(xi_ref, xj_ref, w1_ref, w2_ref, b_ref, m_ref):
    xi = xi_ref[...].astype(jnp.bfloat16)
    xj = xj_ref[...].astype(jnp.bfloat16)
    w1 = w1_ref[...].astype(jnp.bfloat16)
    w2 = w2_ref[...].astype(jnp.bfloat16)
    acc = jnp.dot(xi, w1, preferred_element_type=jnp.float32)
    acc = acc + jnp.dot(xj, w2, preferred_element_type=jnp.float32)
    m_ref[...] = jnp.maximum(acc + b_ref[...], 0.0)


def _scatter_body(m_hbm, rec_hbm, out_hbm, idx_all, rows_a, rows_b, rows_c,
                  zbuf, accum, sem_a, sem_b, sem_c, sem_sa, sem_sb, sem_sc):
    c = lax.axis_index("c")
    s = lax.axis_index("s")
    wid = c * _NS + s
    d = rows_a.shape[1]
    epw = m_hbm.shape[0] // (_NC * _NS)
    ebase = wid * epw
    nch = epw // _CH

    def _rstart(j, rows_v, sem):
        o = ebase + j * _CH
        pltpu.make_async_copy(m_hbm.at[pl.ds(o, _CH)], rows_v, sem).start()

    def _rwait(rows_v, sem):
        pltpu.make_async_copy(m_hbm.at[pl.ds(0, _CH)], rows_v, sem).wait()

    def _scat_start(j, rows_v, sem):
        pltpu.make_async_copy(rows_v, accum.at[idx_all.at[j]],
                              sem).start(add=True)

    def _scat_wait(rows_v, sem):
        pltpu.make_async_copy(rows_v, accum.at[idx_all.at[0]], sem).wait()

    _rstart(0, rows_a, sem_a)
    _rstart(1, rows_b, sem_b)
    pltpu.sync_copy(rec_hbm.at[wid], idx_all)

    def _zrow(t, carry):
        zbuf[t // (d // 16), pl.ds((t % (d // 16)) * 16, 16)] = jnp.zeros(
            (16,), jnp.float32)
        return carry

    lax.fori_loop(0, 16 * (d // 16), _zrow, 0)

    nzc = _N // 16

    def _zacc(i, carry):
        pltpu.sync_copy(zbuf, accum.at[pl.ds((i * _NS + s) * 16, 16)])
        return carry

    lax.fori_loop(0, nzc // _NS, _zacc, 0)

    @pl.when(s == 0)
    def _():
        pltpu.sync_copy(zbuf, accum.at[pl.ds((nzc - 1) * 16, 16)])

    plsc.subcore_barrier()

    _rwait(rows_a, sem_a)
    _scat_start(0, rows_a, sem_sa)
    _rstart(2, rows_c, sem_c)
    _rwait(rows_b, sem_b)
    _scat_start(1, rows_b, sem_sb)
    _scat_wait(rows_a, sem_sa)
    _rstart(3, rows_a, sem_a)
    _rwait(rows_c, sem_c)
    _scat_start(2, rows_c, sem_sc)
    _scat_wait(rows_b, sem_sb)
    _rstart(4, rows_b, sem_b)

    def _triple(i, carry):
        j = 3 * i
        _rwait(rows_a, sem_a)
        _scat_start(j, rows_a, sem_sa)
        _scat_wait(rows_c, sem_sc)
        _rstart(j + 2, rows_c, sem_c)
        _rwait(rows_b, sem_b)
        _scat_start(j + 1, rows_b, sem_sb)
        _scat_wait(rows_a, sem_sa)
        _rstart(j + 3, rows_a, sem_a)
        _rwait(rows_c, sem_c)
        _scat_start(j + 2, rows_c, sem_sc)
        _scat_wait(rows_b, sem_sb)
        _rstart(j + 4, rows_b, sem_b)
        return carry

    lax.fori_loop(1, (nch - 2) // 3, _triple, 0)

    _rwait(rows_a, sem_a)
    _scat_start(nch - 2, rows_a, sem_sa)
    _scat_wait(rows_c, sem_sc)
    _rwait(rows_b, sem_b)
    _scat_start(nch - 1, rows_b, sem_sb)
    _scat_wait(rows_a, sem_sa)
    _scat_wait(rows_b, sem_sb)
    plsc.subcore_barrier()

    def _wout(i, carry):
        o = (i * _NS + s) * 16
        pltpu.sync_copy(accum.at[pl.ds(o, 16)],
                        out_hbm.at[c, pl.ds(o, 16)])
        return carry

    lax.fori_loop(0, nzc // _NS, _wout, 0)

    @pl.when(s == 0)
    def _():
        o = (nzc - 1) * 16
        pltpu.sync_copy(accum.at[pl.ds(o, 16)],
                        out_hbm.at[c, pl.ds(o, 16)])


def _combine_body(p_ref, o_ref):
    o_ref[...] = p_ref[0] + p_ref[1]


def kernel(x_i, x_j, recipients, W, b):
    e, d = x_i.shape
    w1 = W[:d]
    w2 = W[d:]
    b2 = b.reshape(1, d)
    epw = e // (_NC * _NS)
    rec3 = recipients.astype(jnp.int32).reshape(_NC * _NS, epw // _CH, _CH)

    bm = 16000
    m = pl.pallas_call(
        _mlp_body,
        grid=(e // bm,),
        in_specs=[
            pl.BlockSpec((bm, d), lambda i: (i, 0)),
            pl.BlockSpec((bm, d), lambda i: (i, 0)),
            pl.BlockSpec((d, d), lambda i: (0, 0)),
            pl.BlockSpec((d, d), lambda i: (0, 0)),
            pl.BlockSpec((1, d), lambda i: (0, 0)),
        ],
        out_specs=pl.BlockSpec((bm, d), lambda i: (i, 0)),
        out_shape=jax.ShapeDtypeStruct((e, d), jnp.float32),
    )(x_i, x_j, w1, w2, b2)

    mesh = plsc.VectorSubcoreMesh(core_axis_name="c", subcore_axis_name="s")
    scatter = functools.partial(
        pl.kernel,
        out_type=jax.ShapeDtypeStruct((_NC, _N, d), jnp.float32),
        mesh=mesh,
        scratch_types=[
            pltpu.VMEM((epw // _CH, _CH), jnp.int32),
            pltpu.VMEM((_CH, d), jnp.float32),
            pltpu.VMEM((_CH, d), jnp.float32),
            pltpu.VMEM((_CH, d), jnp.float32),
            pltpu.VMEM((16, d), jnp.float32),
            pltpu.VMEM_SHARED((_N, d), jnp.float32),
            pltpu.SemaphoreType.DMA,
            pltpu.SemaphoreType.DMA,
            pltpu.SemaphoreType.DMA,
            pltpu.SemaphoreType.DMA,
            pltpu.SemaphoreType.DMA,
            pltpu.SemaphoreType.DMA,
        ],
    )(_scatter_body)
    partials = scatter(m, rec3)

    aggr = pl.pallas_call(
        _combine_body,
        out_shape=jax.ShapeDtypeStruct((_N, d), jnp.float32),
    )(partials)

    return (aggr, m)
